# Initial kernel scaffold; baseline (speedup 1.0000x reference)
import jax
import jax.numpy as jnp
from jax.experimental import pallas as pl


def kernel(output, anchors):
    raise NotImplementedError("write your pallas kernel here")


# speedup vs baseline: 1.9413x; 1.9413x over previous
import jax
import jax.numpy as jnp
from jax.experimental import pallas as pl
from jax.experimental.pallas import tpu as pltpu

_A = 3
_C = 85
_NET = 608.0


def _decode_kernel(anchors_ref, x_ref, o_ref):
    a = pl.program_id(1)
    x = x_ref[0, 0]
    _, H, W = x_ref.shape[2], x_ref.shape[3], x_ref.shape[4]
    shape = (x_ref.shape[2], H, W)
    cid = jax.lax.broadcasted_iota(jnp.int32, shape, 0)
    gi = jax.lax.broadcasted_iota(jnp.int32, shape, 2).astype(jnp.float32)
    gj = jax.lax.broadcasted_iota(jnp.int32, shape, 1).astype(jnp.float32)
    s = jax.nn.sigmoid(x)
    e = jnp.exp(x)
    aw = anchors_ref[a, 0] * (1.0 / _NET)
    ah = anchors_ref[a, 1] * (1.0 / _NET)
    y = jnp.where(cid == 0, (s + gi) * (1.0 / W),
        jnp.where(cid == 1, (s + gj) * (1.0 / H),
        jnp.where(cid == 2, e * aw,
        jnp.where(cid == 3, e * ah, s))))
    y2 = y.reshape(x_ref.shape[2], H * W)
    o_ref[0, 0] = y2.T


def kernel(output, anchors):
    B, CC, H, W = output.shape
    assert CC == _A * _C
    x5 = output.reshape(B, _A, _C, H, W)
    out = pl.pallas_call(
        _decode_kernel,
        grid=(B, _A),
        in_specs=[
            pl.BlockSpec(memory_space=pltpu.SMEM),
            pl.BlockSpec((1, 1, _C, H, W), lambda b, a: (b, a, 0, 0, 0)),
        ],
        out_specs=pl.BlockSpec((1, 1, H * W, _C), lambda b, a: (b, a, 0, 0)),
        out_shape=jax.ShapeDtypeStruct((B, _A, H * W, _C), jnp.float32),
        compiler_params=pltpu.CompilerParams(
            dimension_semantics=("parallel", "parallel")
        ),
    )(anchors, x5)
    return out.reshape(B, _A * H * W, _C)


# speedup vs baseline: 3.4480x; 1.7762x over previous
import jax
import jax.numpy as jnp
from jax.experimental import pallas as pl
from jax.experimental.pallas import tpu as pltpu

_A = 3
_C = 85
_NET = 608.0


def _decode_kernel(anchors_ref, x_ref, o_ref):
    a = pl.program_id(1)
    x = x_ref[0]
    _, H, W = x_ref.shape[1], x_ref.shape[2], x_ref.shape[3]
    shape = (x_ref.shape[1], H, W)
    cid = jax.lax.broadcasted_iota(jnp.int32, shape, 0)
    gi = jax.lax.broadcasted_iota(jnp.int32, shape, 2).astype(jnp.float32)
    gj = jax.lax.broadcasted_iota(jnp.int32, shape, 1).astype(jnp.float32)
    s = jax.nn.sigmoid(x)
    e = jnp.exp(x)
    aw = anchors_ref[a, 0] * (1.0 / _NET)
    ah = anchors_ref[a, 1] * (1.0 / _NET)
    y = jnp.where(cid == 0, (s + gi) * (1.0 / W),
        jnp.where(cid == 1, (s + gj) * (1.0 / H),
        jnp.where(cid == 2, e * aw,
        jnp.where(cid == 3, e * ah, s))))
    y2 = y.reshape(x_ref.shape[1], H * W)
    o_ref[0] = y2.T


def kernel(output, anchors):
    B, CC, H, W = output.shape
    assert CC == _A * _C
    out = pl.pallas_call(
        _decode_kernel,
        grid=(B, _A),
        in_specs=[
            pl.BlockSpec(memory_space=pltpu.SMEM),
            pl.BlockSpec((1, _C, H, W), lambda b, a: (b, a, 0, 0)),
        ],
        out_specs=pl.BlockSpec((1, H * W, _C), lambda b, a: (b, a, 0)),
        out_shape=jax.ShapeDtypeStruct((B, _A * H * W, _C), jnp.float32),
        compiler_params=pltpu.CompilerParams(
            dimension_semantics=("parallel", "parallel")
        ),
    )(anchors, output)
    return out


# speedup vs baseline: 6.1356x; 1.7794x over previous
import jax
import jax.numpy as jnp
from jax.experimental import pallas as pl
from jax.experimental.pallas import tpu as pltpu

_A = 3
_C = 85
_NET = 608.0
_HP = 2


def _decode_kernel(anchors_ref, x_ref, o_ref):
    hp = pl.program_id(0)
    x = x_ref[...]
    _, W, B, CC = x_ref.shape
    shape = x_ref.shape
    cg = jax.lax.broadcasted_iota(jnp.int32, shape, 3)
    cl = cg % _C
    ai = cg // _C
    gi = jax.lax.broadcasted_iota(jnp.int32, shape, 1).astype(jnp.float32)
    gj = (hp * _HP + jax.lax.broadcasted_iota(jnp.int32, shape, 0)).astype(
        jnp.float32
    )
    s = jax.nn.sigmoid(x)
    e = jnp.exp(x)
    aw = jnp.where(
        ai == 0,
        anchors_ref[0, 0],
        jnp.where(ai == 1, anchors_ref[1, 0], anchors_ref[2, 0]),
    ) * (1.0 / _NET)
    ah = jnp.where(
        ai == 0,
        anchors_ref[0, 1],
        jnp.where(ai == 1, anchors_ref[1, 1], anchors_ref[2, 1]),
    ) * (1.0 / _NET)
    y = jnp.where(cl == 0, (s + gi) * (1.0 / W),
        jnp.where(cl == 1, (s + gj) * (1.0 / W),
        jnp.where(cl == 2, e * aw,
        jnp.where(cl == 3, e * ah, s))))
    for a in range(_A):
        ya = y[:, :, :, a * _C:(a + 1) * _C]
        za = jnp.transpose(ya, (2, 0, 1, 3))
        o_ref[:, a, :, :] = za.reshape(B, _HP * W, _C)


def kernel(output, anchors):
    B, CC, H, W = output.shape
    assert CC == _A * _C
    xt = jnp.transpose(output, (2, 3, 0, 1))
    out = pl.pallas_call(
        _decode_kernel,
        grid=(H // _HP,),
        in_specs=[
            pl.BlockSpec(memory_space=pltpu.SMEM),
            pl.BlockSpec((_HP, W, B, CC), lambda hp: (hp, 0, 0, 0)),
        ],
        out_specs=pl.BlockSpec(
            (B, _A, _HP * W, _C), lambda hp: (0, 0, hp, 0)
        ),
        out_shape=jax.ShapeDtypeStruct((B, _A, H * W, _C), jnp.float32),
        compiler_params=pltpu.CompilerParams(
            dimension_semantics=("arbitrary",)
        ),
    )(anchors, xt)
    return out.reshape(B, _A * H * W, _C)


# speedup vs baseline: 6.5515x; 1.0678x over previous
import jax
import jax.numpy as jnp
from jax.experimental import pallas as pl
from jax.experimental.pallas import tpu as pltpu

_A = 3
_C = 85
_NET = 608.0
_HP = 2


def _decode_kernel(anchors_ref, x_ref, o_ref):
    hp = pl.program_id(0)
    x = x_ref[...]
    HP, W, B, CC = x_ref.shape

    lshape = (1, 1, 1, CC)
    cg = jax.lax.broadcasted_iota(jnp.int32, lshape, 3)
    cl = cg % _C
    ai = cg // _C
    m23 = (cl == 2) | (cl == 3)
    is2 = cl == 2
    awh = jnp.where(
        ai == 0,
        jnp.where(is2, anchors_ref[0, 0], anchors_ref[0, 1]),
        jnp.where(
            ai == 1,
            jnp.where(is2, anchors_ref[1, 0], anchors_ref[1, 1]),
            jnp.where(is2, anchors_ref[2, 0], anchors_ref[2, 1]),
        ),
    ) * (1.0 / _NET)
    sc = jnp.where(cl < 2, 1.0 / W, 1.0)
    m0 = (cl == 0).astype(jnp.float32)
    m1 = (cl == 1).astype(jnp.float32)

    gshape = (HP, W, 1, CC)
    gi = jax.lax.broadcasted_iota(jnp.int32, gshape, 1).astype(jnp.float32)
    gj = (hp * HP + jax.lax.broadcasted_iota(jnp.int32, gshape, 0)).astype(
        jnp.float32
    )
    g = gi * m0 + gj * m1

    s = jax.nn.sigmoid(x)
    e = jnp.exp(x)
    y = jnp.where(m23, e * awh, (s + g) * sc)
    for a in range(_A):
        ya = y[:, :, :, a * _C:(a + 1) * _C]
        za = jnp.transpose(ya, (2, 0, 1, 3))
        o_ref[:, a, :, :] = za.reshape(B, _HP * W, _C)


def kernel(output, anchors):
    B, CC, H, W = output.shape
    assert CC == _A * _C
    xt = jnp.transpose(output, (2, 3, 0, 1))
    out = pl.pallas_call(
        _decode_kernel,
        grid=(H // _HP,),
        in_specs=[
            pl.BlockSpec(memory_space=pltpu.SMEM),
            pl.BlockSpec((_HP, W, B, CC), lambda hp: (hp, 0, 0, 0)),
        ],
        out_specs=pl.BlockSpec(
            (B, _A, _HP * W, _C), lambda hp: (0, 0, hp, 0)
        ),
        out_shape=jax.ShapeDtypeStruct((B, _A, H * W, _C), jnp.float32),
        compiler_params=pltpu.CompilerParams(
            dimension_semantics=("arbitrary",)
        ),
    )(anchors, xt)
    return out.reshape(B, _A * H * W, _C)
